# baseline (speedup 1.0000x reference)
import functools

import jax
import jax.numpy as jnp
from jax import lax
from jax.experimental import pallas as pl
from jax.experimental.pallas import tpu as pltpu
from jax.experimental.pallas import tpu_sc as plsc

_N = 100000
_HIDDEN = 128
_NC = 2
_NS = 16
_NW = _NC * _NS
_CHUNK = 128
_BIG = 3128
_SMALL = 3120
_N_BIG = 20
_N_FULL = 24
_TAIL_BIG = _BIG - _N_FULL * _CHUNK
_TAIL_SMALL = _SMALL - _N_FULL * _CHUNK
_PBUF = 3
_REP = 8
_IDXBUF = 3136


def _pipeline(table_hbm, out_hbm, idx_v, rows, tailbuf, gsems, wsems, tsems, base, tail):
    n_pairs = _N_FULL // 2
    ga = [None] * _PBUF
    gb = [None] * _PBUF
    wc = [None] * _PBUF
    gt = pltpu.async_copy(
        table_hbm.at[idx_v.at[pl.ds(_N_FULL * _CHUNK, tail)]],
        tailbuf.at[pl.ds(0, tail)],
        tsems[0],
    )
    for p in range(n_pairs + _PBUF - 1):
        if p < n_pairs:
            b = p % _PBUF
            if p >= _PBUF:
                wc[b].wait()
            ga[b] = pltpu.async_copy(
                table_hbm.at[idx_v.at[pl.ds(2 * p * _CHUNK, _CHUNK)]],
                rows[b].at[pl.ds(0, _CHUNK)],
                gsems[b],
            )
            gb[b] = pltpu.async_copy(
                table_hbm.at[idx_v.at[pl.ds((2 * p + 1) * _CHUNK, _CHUNK)]],
                rows[b].at[pl.ds(_CHUNK, _CHUNK)],
                gsems[b],
            )
        q = p - (_PBUF - 1)
        if 0 <= q < n_pairs:
            b = q % _PBUF
            ga[b].wait()
            gb[b].wait()
            wc[b] = pltpu.async_copy(
                rows[b],
                out_hbm.at[pl.ds(base + q * 2 * _CHUNK, 2 * _CHUNK)],
                wsems[b],
            )
            if q == 0:
                gt.wait()
                wt = pltpu.async_copy(
                    tailbuf.at[pl.ds(0, tail)],
                    out_hbm.at[pl.ds(base + _N_FULL * _CHUNK, tail)],
                    tsems[1],
                )
    for q in range(max(0, n_pairs - _PBUF), n_pairs):
        wc[q % _PBUF].wait()
    wt.wait()


def _body(idx_hbm, table_hbm, out_hbm, idx_v, *bufs):
    rows = bufs[:_PBUF]
    tailbuf = bufs[_PBUF]
    gsems = bufs[_PBUF + 1 : 2 * _PBUF + 1]
    wsems = bufs[2 * _PBUF + 1 : 3 * _PBUF + 1]
    tsems = bufs[3 * _PBUF + 1 :]
    wid = lax.axis_index("s") * _NC + lax.axis_index("c")
    is_big = wid < _N_BIG
    base = jnp.where(is_big, wid * _BIG, _N_BIG * _BIG + (wid - _N_BIG) * _SMALL)
    base = pl.multiple_of(base, 8)

    @pl.when(is_big)
    def _():
        pltpu.sync_copy(idx_hbm.at[pl.ds(base, _BIG)], idx_v.at[pl.ds(0, _BIG)])

    @pl.when(jnp.logical_not(is_big))
    def _():
        pltpu.sync_copy(idx_hbm.at[pl.ds(base, _SMALL)], idx_v.at[pl.ds(0, _SMALL)])

    off = jnp.arange(16, dtype=jnp.int32) % _REP * 21 + wid * (_REP * 21)
    for s in range(_IDXBUF // 16):
        sl = pl.ds(s * 16, 16)
        idx_v[sl] = idx_v[sl] + off

    @pl.when(is_big)
    def _():
        _pipeline(
            table_hbm, out_hbm, idx_v, rows, tailbuf, gsems, wsems, tsems,
            base, _TAIL_BIG,
        )

    @pl.when(jnp.logical_not(is_big))
    def _():
        _pipeline(
            table_hbm, out_hbm, idx_v, rows, tailbuf, gsems, wsems, tsems,
            base, _TAIL_SMALL,
        )


@jax.jit
def _lookup(idx, table):
    mesh = plsc.VectorSubcoreMesh(
        core_axis_name="c", subcore_axis_name="s", num_cores=_NC, num_subcores=_NS
    )
    run = functools.partial(
        pl.kernel,
        out_type=jax.ShapeDtypeStruct((_N, _HIDDEN), jnp.float32),
        mesh=mesh,
        scratch_types=(
            [pltpu.VMEM((_IDXBUF,), jnp.int32)]
            + [pltpu.VMEM((2 * _CHUNK, _HIDDEN), jnp.float32)] * _PBUF
            + [pltpu.VMEM((_CHUNK, _HIDDEN), jnp.float32)]
            + [pltpu.SemaphoreType.DMA] * (2 * _PBUF + 2)
        ),
    )(_body)
    return run(idx, table)


def kernel(x, emb_weight):
    table_rep = jnp.tile(emb_weight.astype(jnp.float32), (_NW * _REP, 1))
    return _lookup(x.astype(jnp.int32), table_rep)
